# baseline (speedup 1.0000x reference)
import functools

import jax
import jax.numpy as jnp
from jax import lax
from jax.experimental import pallas as pl
from jax.experimental.pallas import tpu as pltpu
from jax.experimental.pallas import tpu_sc as plsc

CH = 128
NB = 4
RB = 80
NSUB = 16


def _mesh():
    return plsc.VectorSubcoreMesh(core_axis_name="c", subcore_axis_name="s")


_SC_PARAMS = pltpu.CompilerParams(use_tc_tiling_on_sc=False)


def _tc_rel_tables(rel_emb, W_out, b_out, W_in, b_in):
    def body(re_ref, wo_ref, bo_ref, wi_ref, bi_ref, out_ref):
        re = re_ref[:]
        ro = jnp.dot(re, wo_ref[:], preferred_element_type=jnp.float32) + bo_ref[:]
        ri = jnp.dot(re, wi_ref[:], preferred_element_type=jnp.float32) + bi_ref[:]
        ones = jnp.ones((16, 1), jnp.float32)
        zpad = jnp.zeros((16, 15), jnp.float32)
        out_ref[:] = jnp.concatenate(
            [jnp.concatenate([ro, ones, zpad], axis=1),
             jnp.concatenate([ri, ones, zpad], axis=1)], axis=0)

    return pl.pallas_call(
        body, out_shape=jax.ShapeDtypeStruct((32, 48), jnp.float32),
    )(rel_emb, W_out, b_out.reshape(1, -1), W_in, b_in.reshape(1, -1))


def _tc_build_x(input_feat, nd_cat, d1_cat, n):
    B = 1000
    grid = n // B

    def body(f_ref, ndo_ref, ndi_ref, d1o_ref, d1i_ref,
             xa_ref, xb_ref, nd0_ref, ns1_ref, nd1_ref):
        ndo = ndo_ref[:]
        ndi = ndi_ref[:]
        deg_o = ndo[:, 32:33]
        deg_i = ndi[:, 32:33]
        ns0 = jnp.where(deg_o > 0, lax.rsqrt(deg_o), 0.0)
        nd0_ref[:] = jnp.where(deg_i > 0, lax.rsqrt(deg_i), 0.0)
        d1o = d1o_ref[:, 0:1]
        d1i = d1i_ref[:, 0:1]
        ns1_ref[:] = jnp.where(d1o > 0, lax.rsqrt(d1o), 0.0)
        nd1_ref[:] = jnp.where(d1i > 0, lax.rsqrt(d1i), 0.0)
        rel = (ndo[:, :32] + ndi[:, :32]) * ns0
        feat = f_ref[:] * ns0
        xa_ref[:] = feat[:, :80]
        xb_ref[:] = jnp.concatenate([feat[:, 80:], rel], axis=1)

    f32 = jnp.float32
    return pl.pallas_call(
        body,
        grid=(grid,),
        in_specs=[
            pl.BlockSpec((B, 128), lambda i: (i, 0)),
            pl.BlockSpec((B, 48), lambda i: (i, 0)),
            pl.BlockSpec((B, 48), lambda i, g=grid: (i + g, 0)),
            pl.BlockSpec((B, 16), lambda i: (i, 0)),
            pl.BlockSpec((B, 16), lambda i, g=grid: (i + g, 0)),
        ],
        out_specs=[
            pl.BlockSpec((B, 80), lambda i: (i, 0)),
            pl.BlockSpec((B, 80), lambda i: (i, 0)),
            pl.BlockSpec((B, 1), lambda i: (i, 0)),
            pl.BlockSpec((B, 1), lambda i: (i, 0)),
            pl.BlockSpec((B, 1), lambda i: (i, 0)),
        ],
        out_shape=[
            jax.ShapeDtypeStruct((n, 80), f32),
            jax.ShapeDtypeStruct((n, 80), f32),
            jax.ShapeDtypeStruct((n, 1), f32),
            jax.ShapeDtypeStruct((n, 1), f32),
            jax.ShapeDtypeStruct((n, 1), f32),
        ],
    )(input_feat, nd_cat, nd_cat, d1_cat, d1_cat)


def _tc_layer(agg_cat, ndst, nsrc_next, W, b, n, W2, H, scale_out):
    B = 1000
    grid = n // B

    def body(aa_ref, ab_ref, nd_ref, ns_ref, w_ref, b_ref, oa_ref, ob_ref):
        h = (jnp.dot(aa_ref[:], w_ref[:W2, :],
                     preferred_element_type=jnp.float32)
             + jnp.dot(ab_ref[:], w_ref[W2:, :],
                       preferred_element_type=jnp.float32))
        x = jnp.maximum(h * nd_ref[:] + b_ref[:], 0.0)
        if scale_out:
            x = x * ns_ref[:]
        oa_ref[:] = x[:, : H // 2]
        ob_ref[:] = x[:, H // 2:]

    f32 = jnp.float32
    return pl.pallas_call(
        body,
        grid=(grid,),
        in_specs=[
            pl.BlockSpec((B, W2), lambda i: (i, 0)),
            pl.BlockSpec((B, W2), lambda i, g=grid: (i + g, 0)),
            pl.BlockSpec((B, 1), lambda i: (i, 0)),
            pl.BlockSpec((B, 1), lambda i: (i, 0)),
            pl.BlockSpec((2 * W2, H), lambda i: (0, 0)),
            pl.BlockSpec((1, H), lambda i: (0, 0)),
        ],
        out_specs=[
            pl.BlockSpec((B, H // 2), lambda i: (i, 0)),
            pl.BlockSpec((B, H // 2), lambda i: (i, 0)),
        ],
        out_shape=[
            jax.ShapeDtypeStruct((n, H // 2), f32),
            jax.ShapeDtypeStruct((n, H // 2), f32),
        ],
    )(agg_cat, agg_cat, ndst, nsrc_next, W, b.reshape(1, -1))


def _tc_fc(x_cat, Wcat, bcat, n):
    B = 1000
    grid = n // B

    def body(aa_ref, ab_ref, w_ref, b_ref, oa_ref, ob_ref):
        y = (jnp.dot(aa_ref[:], w_ref[:128, :],
                     preferred_element_type=jnp.float32)
             + jnp.dot(ab_ref[:], w_ref[128:, :],
                       preferred_element_type=jnp.float32)) + b_ref[:]
        oa_ref[:] = y[:, :128]
        ob_ref[:] = y[:, 128:]

    f32 = jnp.float32
    return pl.pallas_call(
        body,
        grid=(grid,),
        in_specs=[
            pl.BlockSpec((B, 128), lambda i: (i, 0)),
            pl.BlockSpec((B, 128), lambda i, g=grid: (i + g, 0)),
            pl.BlockSpec((256, 256), lambda i: (0, 0)),
            pl.BlockSpec((1, 256), lambda i: (0, 0)),
        ],
        out_specs=[
            pl.BlockSpec((B, 128), lambda i: (i, 0)),
            pl.BlockSpec((B, 128), lambda i: (i, 0)),
        ],
        out_shape=[
            jax.ShapeDtypeStruct((n, 128), f32),
            jax.ShapeDtypeStruct((n, 128), f32),
        ],
    )(x_cat, x_cat, Wcat, bcat.reshape(1, -1))


def _sc_embed(T, etype, nodes0, nodes1, n, E):
    nblk = E // (CH * NB)
    nzb = n // RB
    zeros48 = jnp.zeros((RB, 48), jnp.float32)
    zeros16 = jnp.zeros((RB, 16), jnp.float32)
    ones16 = jnp.ones((CH, 16), jnp.float32)

    @functools.partial(
        pl.kernel,
        out_type=(jax.ShapeDtypeStruct((2 * n, 48), jnp.float32),
                  jax.ShapeDtypeStruct((2 * n, 16), jnp.float32)),
        mesh=_mesh(),
        compiler_params=_SC_PARAMS,
        scratch_types=[
            [pltpu.VMEM((CH,), jnp.int32) for _ in range(NB)],
            [pltpu.VMEM((CH,), jnp.int32) for _ in range(NB)],
            [pltpu.VMEM((CH,), jnp.int32) for _ in range(NB)],
            pltpu.VMEM((NB, CH, 48), jnp.float32),
            pltpu.VMEM((CH, 16), jnp.float32),
            pltpu.VMEM_SHARED((n, 48), jnp.float32),
            pltpu.VMEM_SHARED((n, 16), jnp.float32),
            pltpu.SemaphoreType.DMA((3, NB)),
            pltpu.SemaphoreType.DMA((NB,)),
            pltpu.SemaphoreType.DMA((NB,)),
            pltpu.SemaphoreType.DMA((NB,)),
        ],
    )
    def k(T_h, et_h, n0_h, n1_h, z48_h, z16_h, o16_h, outA_h, outB_h,
          tbufs, nbufs, n1bufs, rows, onesb, accA, accB,
          semI, semG, semS, semO):
        c = lax.axis_index("c")
        s = lax.axis_index("s")

        @pl.loop(s, nzb, step=NSUB)
        def _(b):
            pltpu.sync_copy(z48_h, accA.at[pl.ds(b * RB, RB)])
            pltpu.sync_copy(z16_h, accB.at[pl.ds(b * RB, RB)])

        pltpu.sync_copy(o16_h, onesb)
        plsc.subcore_barrier()

        eoff = c * E
        toff = c * 16

        @pl.loop(s, nblk, step=NSUB)
        def _(blk):
            base = blk * (NB * CH)
            ld = []
            for j in range(NB):
                ld.append(pltpu.async_copy(
                    et_h.at[pl.ds(base + j * CH, CH)], tbufs[j],
                    semI.at[0, j]))
                ld.append(pltpu.async_copy(
                    n0_h.at[pl.ds(eoff + base + j * CH, CH)], nbufs[j],
                    semI.at[1, j]))
                ld.append(pltpu.async_copy(
                    n1_h.at[pl.ds(eoff + base + j * CH, CH)], n1bufs[j],
                    semI.at[2, j]))
            for d in ld:
                d.wait()
            for j in range(NB):
                for i in range(CH // 16):
                    tbufs[j][pl.ds(i * 16, 16)] = (
                        tbufs[j][pl.ds(i * 16, 16)] + toff)
            gd = [pltpu.async_copy(T_h.at[tbufs[j]], rows.at[j], semG.at[j])
                  for j in range(NB)]
            sd = []
            for j in range(NB):
                gd[j].wait()
                sd.append(pltpu.async_copy(rows.at[j], accA.at[nbufs[j]],
                                           semS.at[j], add=True))
                sd.append(pltpu.async_copy(onesb, accB.at[n1bufs[j]],
                                           semO.at[j], add=True))
            for d in sd:
                d.wait()

        plsc.subcore_barrier()
        noff = c * n

        @pl.loop(s, nzb, step=NSUB)
        def _(b):
            pltpu.sync_copy(accA.at[pl.ds(b * RB, RB)],
                            outA_h.at[pl.ds(noff + b * RB, RB)])
            pltpu.sync_copy(accB.at[pl.ds(b * RB, RB)],
                            outB_h.at[pl.ds(noff + b * RB, RB)])

    return k(T, etype, nodes0, nodes1, zeros48, zeros16, ones16)


def _sc_conv(xs_cat, src, dst, n, E, W2, NB=NB, ch=CH):
    nblk = E // (ch * NB)
    nzb = n // RB
    zerosW = jnp.zeros((RB, W2), jnp.float32)

    @functools.partial(
        pl.kernel,
        out_type=jax.ShapeDtypeStruct((2 * n, W2), jnp.float32),
        mesh=_mesh(),
        compiler_params=_SC_PARAMS,
        scratch_types=[
            [pltpu.VMEM((ch,), jnp.int32) for _ in range(NB)],
            [pltpu.VMEM((ch,), jnp.int32) for _ in range(NB)],
            pltpu.VMEM((NB, ch, W2), jnp.float32),
            pltpu.VMEM_SHARED((n, W2), jnp.float32),
            pltpu.SemaphoreType.DMA((2, NB)),
            pltpu.SemaphoreType.DMA((NB,)),
            pltpu.SemaphoreType.DMA((NB,)),
        ],
    )
    def k(xs_h, src_h, dst_h, zW_h, out_h, sbufs, dbufs, rows, acc,
          semI, semG, semS):
        c = lax.axis_index("c")
        s = lax.axis_index("s")

        @pl.loop(s, nzb, step=NSUB)
        def _(b):
            pltpu.sync_copy(zW_h, acc.at[pl.ds(b * RB, RB)])

        plsc.subcore_barrier()
        noff = c * n

        @pl.loop(s, nblk, step=NSUB)
        def _(blk):
            base = blk * (NB * ch)
            ld = []
            for j in range(NB):
                ld.append(pltpu.async_copy(
                    src_h.at[pl.ds(base + j * ch, ch)], sbufs[j],
                    semI.at[0, j]))
                ld.append(pltpu.async_copy(
                    dst_h.at[pl.ds(base + j * ch, ch)], dbufs[j],
                    semI.at[1, j]))
            for d in ld:
                d.wait()
            for j in range(NB):
                for i in range(ch // 16):
                    sbufs[j][pl.ds(i * 16, 16)] = (
                        sbufs[j][pl.ds(i * 16, 16)] + noff)
            gd = [pltpu.async_copy(xs_h.at[sbufs[j]], rows.at[j], semG.at[j])
                  for j in range(NB)]
            sd = []
            for j in range(NB):
                gd[j].wait()
                sd.append(pltpu.async_copy(rows.at[j], acc.at[dbufs[j]],
                                           semS.at[j], add=True))
            for d in sd:
                d.wait()

        plsc.subcore_barrier()

        @pl.loop(s, nzb, step=NSUB)
        def _(b):
            pltpu.sync_copy(acc.at[pl.ds(b * RB, RB)],
                            out_h.at[pl.ds(noff + b * RB, RB)])

    return k(xs_cat, src, dst, zerosW)


def _sc_final(Ycat, ssrc, sdst, n, ES):
    NF = 2
    nblk = ES // (CH * NF)

    @functools.partial(
        pl.kernel,
        out_type=jax.ShapeDtypeStruct((ES, 128), jnp.float32),
        mesh=_mesh(),
        compiler_params=_SC_PARAMS,
        scratch_types=[
            [pltpu.VMEM((CH,), jnp.int32) for _ in range(NF)],
            [pltpu.VMEM((CH,), jnp.int32) for _ in range(NF)],
            pltpu.VMEM((NF, CH, 128), jnp.float32),
            pltpu.VMEM((NF, CH, 128), jnp.float32),
            pltpu.SemaphoreType.DMA((2, NF)),
            pltpu.SemaphoreType.DMA((NF,)),
            pltpu.SemaphoreType.DMA((NF,)),
            pltpu.SemaphoreType.DMA((NF,)),
        ],
    )
    def k(Y_h, s_h, d_h, out_h, abufs, bbufs, A, B, semI, semA, semB, semS):
        c = lax.axis_index("c")
        s = lax.axis_index("s")
        wid = s * 2 + c

        @pl.loop(wid, nblk, step=2 * NSUB)
        def _(blk):
            base = blk * (NF * CH)
            ld = []
            for j in range(NF):
                ld.append(pltpu.async_copy(
                    s_h.at[pl.ds(base + j * CH, CH)], abufs[j],
                    semI.at[0, j]))
                ld.append(pltpu.async_copy(
                    d_h.at[pl.ds(base + j * CH, CH)], bbufs[j],
                    semI.at[1, j]))
            for d in ld:
                d.wait()
            for j in range(NF):
                for i in range(CH // 16):
                    bbufs[j][pl.ds(i * 16, 16)] = (
                        bbufs[j][pl.ds(i * 16, 16)] + n)
            gda = [pltpu.async_copy(Y_h.at[abufs[j]], A.at[j], semA.at[j])
                   for j in range(NF)]
            gdb = [pltpu.async_copy(Y_h.at[bbufs[j]], B.at[j], semB.at[j])
                   for j in range(NF)]
            sd = []
            for j in range(NF):
                gda[j].wait()
                gdb[j].wait()

                @pl.loop(0, CH)
                def _(r):
                    for i in range(8):
                        A[j, r, pl.ds(i * 16, 16)] = (
                            A[j, r, pl.ds(i * 16, 16)]
                            + B[j, r, pl.ds(i * 16, 16)])

                sd.append(pltpu.async_copy(
                    A.at[j], out_h.at[pl.ds(base + j * CH, CH)], semS.at[j]))
            for d in sd:
                d.wait()

    return k(Ycat, ssrc, sdst)


def kernel(input_feat, edge_index0, edge_index1, edge_type, edge_subg_index,
           rel_emb, W_out, b_out, W_in, b_in, W_g0, b_g0, W_g1, b_g1,
           W_fc, b_fc):
    n = input_feat.shape[0]
    E = edge_type.shape[0]
    ES = edge_subg_index.shape[1]

    T = _tc_rel_tables(rel_emb, W_out, b_out, W_in, b_in)
    nodes0 = edge_index0.reshape(-1)
    nodes1 = edge_index1.reshape(-1)
    nd_cat, d1_cat = _sc_embed(T, edge_type, nodes0, nodes1, n, E)

    xa, xb, nd0, ns1, nd1 = _tc_build_x(input_feat, nd_cat, d1_cat, n)
    xs_cat = jnp.concatenate([xa, xb], axis=0)

    agg0 = _sc_conv(xs_cat, edge_index0[0], edge_index0[1], n, E, 80, NB=2, ch=320)
    x1a, x1b = _tc_layer(agg0, nd0, ns1, W_g0, b_g0, n, 80, 256, True)
    x1s_cat = jnp.concatenate([x1a, x1b], axis=0)

    agg1 = _sc_conv(x1s_cat, edge_index1[0], edge_index1[1], n, E, 128, NB=2)
    x2a, x2b = _tc_layer(agg1, nd1, nd1, W_g1, b_g1, n, 128, 256, False)
    x2_cat = jnp.concatenate([x2a, x2b], axis=0)

    Wcat = jnp.concatenate([W_fc[:256], W_fc[256:]], axis=1)
    bcat = jnp.concatenate([b_fc, jnp.zeros_like(b_fc)], axis=0)
    Yt, Yb = _tc_fc(x2_cat, Wcat, bcat, n)
    Ycat = jnp.concatenate([Yt, Yb], axis=0)

    return _sc_final(Ycat, edge_subg_index[0], edge_subg_index[1], n, ES)


# speedup vs baseline: 1.6522x; 1.6522x over previous
import functools

import jax
import jax.numpy as jnp
from jax import lax
from jax.experimental import pallas as pl
from jax.experimental.pallas import tpu as pltpu
from jax.experimental.pallas import tpu_sc as plsc

CH = 128
NB = 4
RB = 80
NSUB = 16


def _mesh():
    return plsc.VectorSubcoreMesh(core_axis_name="c", subcore_axis_name="s")


_SC_PARAMS = pltpu.CompilerParams(use_tc_tiling_on_sc=False)


def _tc_rel_tables(rel_emb, W_out, b_out, W_in, b_in):
    def body(re_ref, wo_ref, bo_ref, wi_ref, bi_ref, out_ref):
        re = re_ref[:]
        ro = jnp.dot(re, wo_ref[:], preferred_element_type=jnp.float32) + bo_ref[:]
        ri = jnp.dot(re, wi_ref[:], preferred_element_type=jnp.float32) + bi_ref[:]
        ones = jnp.ones((16, 1), jnp.float32)
        zpad = jnp.zeros((16, 15), jnp.float32)
        out_ref[:] = jnp.concatenate(
            [jnp.concatenate([ro, ones, zpad], axis=1),
             jnp.concatenate([ri, ones, zpad], axis=1)], axis=0)

    return pl.pallas_call(
        body, out_shape=jax.ShapeDtypeStruct((32, 48), jnp.float32),
    )(rel_emb, W_out, b_out.reshape(1, -1), W_in, b_in.reshape(1, -1))


def _tc_build_x(input_feat, nd_cat, d1_cat, n):
    B = 1000
    grid = n // B

    def body(f_ref, ndo_ref, ndi_ref, d1o_ref, d1i_ref,
             xa_ref, xb_ref, nd0_ref, ns1_ref, nd1_ref):
        ndo = ndo_ref[:]
        ndi = ndi_ref[:]
        deg_o = ndo[:, 32:33]
        deg_i = ndi[:, 32:33]
        ns0 = jnp.where(deg_o > 0, lax.rsqrt(deg_o), 0.0)
        nd0_ref[:] = jnp.where(deg_i > 0, lax.rsqrt(deg_i), 0.0)
        d1o = d1o_ref[:, 0:1]
        d1i = d1i_ref[:, 0:1]
        ns1_ref[:] = jnp.where(d1o > 0, lax.rsqrt(d1o), 0.0)
        nd1_ref[:] = jnp.where(d1i > 0, lax.rsqrt(d1i), 0.0)
        rel = (ndo[:, :32] + ndi[:, :32]) * ns0
        feat = f_ref[:] * ns0
        xa_ref[:] = feat[:, :80]
        xb_ref[:] = jnp.concatenate([feat[:, 80:], rel], axis=1)

    f32 = jnp.float32
    return pl.pallas_call(
        body,
        grid=(grid,),
        in_specs=[
            pl.BlockSpec((B, 128), lambda i: (i, 0)),
            pl.BlockSpec((B, 48), lambda i: (i, 0)),
            pl.BlockSpec((B, 48), lambda i, g=grid: (i + g, 0)),
            pl.BlockSpec((B, 16), lambda i: (i, 0)),
            pl.BlockSpec((B, 16), lambda i, g=grid: (i + g, 0)),
        ],
        out_specs=[
            pl.BlockSpec((B, 80), lambda i: (i, 0)),
            pl.BlockSpec((B, 80), lambda i: (i, 0)),
            pl.BlockSpec((B, 1), lambda i: (i, 0)),
            pl.BlockSpec((B, 1), lambda i: (i, 0)),
            pl.BlockSpec((B, 1), lambda i: (i, 0)),
        ],
        out_shape=[
            jax.ShapeDtypeStruct((n, 80), f32),
            jax.ShapeDtypeStruct((n, 80), f32),
            jax.ShapeDtypeStruct((n, 1), f32),
            jax.ShapeDtypeStruct((n, 1), f32),
            jax.ShapeDtypeStruct((n, 1), f32),
        ],
    )(input_feat, nd_cat, nd_cat, d1_cat, d1_cat)


def _tc_layer(agg_cat, ndst, nsrc_next, W, b, n, W2, H, scale_out):
    B = 1000
    grid = n // B

    def body(aa_ref, ab_ref, nd_ref, ns_ref, w_ref, b_ref, oa_ref, ob_ref):
        h = (jnp.dot(aa_ref[:], w_ref[:W2, :],
                     preferred_element_type=jnp.float32)
             + jnp.dot(ab_ref[:], w_ref[W2:, :],
                       preferred_element_type=jnp.float32))
        x = jnp.maximum(h * nd_ref[:] + b_ref[:], 0.0)
        if scale_out:
            x = x * ns_ref[:]
        oa_ref[:] = x[:, : H // 2]
        ob_ref[:] = x[:, H // 2:]

    f32 = jnp.float32
    return pl.pallas_call(
        body,
        grid=(grid,),
        in_specs=[
            pl.BlockSpec((B, W2), lambda i: (i, 0)),
            pl.BlockSpec((B, W2), lambda i, g=grid: (i + g, 0)),
            pl.BlockSpec((B, 1), lambda i: (i, 0)),
            pl.BlockSpec((B, 1), lambda i: (i, 0)),
            pl.BlockSpec((2 * W2, H), lambda i: (0, 0)),
            pl.BlockSpec((1, H), lambda i: (0, 0)),
        ],
        out_specs=[
            pl.BlockSpec((B, H // 2), lambda i: (i, 0)),
            pl.BlockSpec((B, H // 2), lambda i: (i, 0)),
        ],
        out_shape=[
            jax.ShapeDtypeStruct((n, H // 2), f32),
            jax.ShapeDtypeStruct((n, H // 2), f32),
        ],
    )(agg_cat, agg_cat, ndst, nsrc_next, W, b.reshape(1, -1))


def _tc_fc(x_cat, Wcat, bcat, n):
    B = 1000
    grid = n // B

    def body(aa_ref, ab_ref, w_ref, b_ref, oa_ref, ob_ref):
        y = (jnp.dot(aa_ref[:], w_ref[:128, :],
                     preferred_element_type=jnp.float32)
             + jnp.dot(ab_ref[:], w_ref[128:, :],
                       preferred_element_type=jnp.float32)) + b_ref[:]
        oa_ref[:] = y[:, :128]
        ob_ref[:] = y[:, 128:]

    f32 = jnp.float32
    return pl.pallas_call(
        body,
        grid=(grid,),
        in_specs=[
            pl.BlockSpec((B, 128), lambda i: (i, 0)),
            pl.BlockSpec((B, 128), lambda i, g=grid: (i + g, 0)),
            pl.BlockSpec((256, 256), lambda i: (0, 0)),
            pl.BlockSpec((1, 256), lambda i: (0, 0)),
        ],
        out_specs=[
            pl.BlockSpec((B, 128), lambda i: (i, 0)),
            pl.BlockSpec((B, 128), lambda i: (i, 0)),
        ],
        out_shape=[
            jax.ShapeDtypeStruct((n, 128), f32),
            jax.ShapeDtypeStruct((n, 128), f32),
        ],
    )(x_cat, x_cat, Wcat, bcat.reshape(1, -1))


def _sc_embed(T, etype, nodes0, nodes1, n, E):
    nblk = E // (CH * NB)
    nzb = n // RB
    zeros48 = jnp.zeros((RB, 48), jnp.float32)
    zeros16 = jnp.zeros((RB, 16), jnp.float32)
    ones16 = jnp.ones((CH, 16), jnp.float32)

    @functools.partial(
        pl.kernel,
        out_type=(jax.ShapeDtypeStruct((2 * n, 48), jnp.float32),
                  jax.ShapeDtypeStruct((2 * n, 16), jnp.float32)),
        mesh=_mesh(),
        compiler_params=_SC_PARAMS,
        scratch_types=[
            [pltpu.VMEM((CH,), jnp.int32) for _ in range(NB)],
            [pltpu.VMEM((CH,), jnp.int32) for _ in range(NB)],
            [pltpu.VMEM((CH,), jnp.int32) for _ in range(NB)],
            pltpu.VMEM((NB, CH, 48), jnp.float32),
            pltpu.VMEM((CH, 16), jnp.float32),
            pltpu.VMEM_SHARED((n, 48), jnp.float32),
            pltpu.VMEM_SHARED((n, 16), jnp.float32),
            pltpu.SemaphoreType.DMA((3, NB)),
            pltpu.SemaphoreType.DMA((NB,)),
            pltpu.SemaphoreType.DMA((NB,)),
            pltpu.SemaphoreType.DMA((NB,)),
        ],
    )
    def k(T_h, et_h, n0_h, n1_h, z48_h, z16_h, o16_h, outA_h, outB_h,
          tbufs, nbufs, n1bufs, rows, onesb, accA, accB,
          semI, semG, semS, semO):
        c = lax.axis_index("c")
        s = lax.axis_index("s")

        @pl.loop(s, nzb, step=NSUB)
        def _(b):
            pltpu.sync_copy(z48_h, accA.at[pl.ds(b * RB, RB)])
            pltpu.sync_copy(z16_h, accB.at[pl.ds(b * RB, RB)])

        pltpu.sync_copy(o16_h, onesb)
        plsc.subcore_barrier()

        eoff = c * E
        toff = c * 16

        @pl.loop(s, nblk, step=NSUB)
        def _(blk):
            base = blk * (NB * CH)
            ld = []
            for j in range(NB):
                ld.append(pltpu.async_copy(
                    et_h.at[pl.ds(base + j * CH, CH)], tbufs[j],
                    semI.at[0, j]))
                ld.append(pltpu.async_copy(
                    n0_h.at[pl.ds(eoff + base + j * CH, CH)], nbufs[j],
                    semI.at[1, j]))
                ld.append(pltpu.async_copy(
                    n1_h.at[pl.ds(eoff + base + j * CH, CH)], n1bufs[j],
                    semI.at[2, j]))
            for d in ld:
                d.wait()
            off16 = lax.iota(jnp.int32, 16) * 32
            for j in range(NB):
                for i in range(CH // 16):
                    tbufs[j][pl.ds(i * 16, 16)] = (
                        tbufs[j][pl.ds(i * 16, 16)] + (toff + 512 * i)
                        + off16)
            gd = [pltpu.async_copy(T_h.at[tbufs[j]], rows.at[j], semG.at[j])
                  for j in range(NB)]
            sd = []
            for j in range(NB):
                gd[j].wait()
                sd.append(pltpu.async_copy(rows.at[j], accA.at[nbufs[j]],
                                           semS.at[j], add=True))
                sd.append(pltpu.async_copy(onesb, accB.at[n1bufs[j]],
                                           semO.at[j], add=True))
            for d in sd:
                d.wait()

        plsc.subcore_barrier()
        noff = c * n

        @pl.loop(s, nzb, step=NSUB)
        def _(b):
            pltpu.sync_copy(accA.at[pl.ds(b * RB, RB)],
                            outA_h.at[pl.ds(noff + b * RB, RB)])
            pltpu.sync_copy(accB.at[pl.ds(b * RB, RB)],
                            outB_h.at[pl.ds(noff + b * RB, RB)])

    return k(T, etype, nodes0, nodes1, zeros48, zeros16, ones16)


def _sc_conv(xs_cat, src, dst, n, E, W2, NB=NB, ch=CH):
    nblk = E // (ch * NB)
    nzb = n // RB
    zerosW = jnp.zeros((RB, W2), jnp.float32)

    @functools.partial(
        pl.kernel,
        out_type=jax.ShapeDtypeStruct((2 * n, W2), jnp.float32),
        mesh=_mesh(),
        compiler_params=_SC_PARAMS,
        scratch_types=[
            [pltpu.VMEM((ch,), jnp.int32) for _ in range(NB)],
            [pltpu.VMEM((ch,), jnp.int32) for _ in range(NB)],
            pltpu.VMEM((NB, ch, W2), jnp.float32),
            pltpu.VMEM_SHARED((n, W2), jnp.float32),
            pltpu.SemaphoreType.DMA((2, NB)),
            pltpu.SemaphoreType.DMA((NB,)),
            pltpu.SemaphoreType.DMA((NB,)),
        ],
    )
    def k(xs_h, src_h, dst_h, zW_h, out_h, sbufs, dbufs, rows, acc,
          semI, semG, semS):
        c = lax.axis_index("c")
        s = lax.axis_index("s")

        @pl.loop(s, nzb, step=NSUB)
        def _(b):
            pltpu.sync_copy(zW_h, acc.at[pl.ds(b * RB, RB)])

        plsc.subcore_barrier()
        noff = c * n

        @pl.loop(s, nblk, step=NSUB)
        def _(blk):
            base = blk * (NB * ch)
            ld = []
            for j in range(NB):
                ld.append(pltpu.async_copy(
                    src_h.at[pl.ds(base + j * ch, ch)], sbufs[j],
                    semI.at[0, j]))
                ld.append(pltpu.async_copy(
                    dst_h.at[pl.ds(base + j * ch, ch)], dbufs[j],
                    semI.at[1, j]))
            for d in ld:
                d.wait()
            for j in range(NB):
                for i in range(ch // 16):
                    sbufs[j][pl.ds(i * 16, 16)] = (
                        sbufs[j][pl.ds(i * 16, 16)] + noff)
            gd = [pltpu.async_copy(xs_h.at[sbufs[j]], rows.at[j], semG.at[j])
                  for j in range(NB)]
            sd = []
            for j in range(NB):
                gd[j].wait()
                sd.append(pltpu.async_copy(rows.at[j], acc.at[dbufs[j]],
                                           semS.at[j], add=True))
            for d in sd:
                d.wait()

        plsc.subcore_barrier()

        @pl.loop(s, nzb, step=NSUB)
        def _(b):
            pltpu.sync_copy(acc.at[pl.ds(b * RB, RB)],
                            out_h.at[pl.ds(noff + b * RB, RB)])

    return k(xs_cat, src, dst, zerosW)


def _sc_final(Ycat, ssrc, sdst, n, ES):
    NF = 2
    nblk = ES // (CH * NF)

    @functools.partial(
        pl.kernel,
        out_type=jax.ShapeDtypeStruct((ES, 128), jnp.float32),
        mesh=_mesh(),
        compiler_params=_SC_PARAMS,
        scratch_types=[
            [pltpu.VMEM((CH,), jnp.int32) for _ in range(NF)],
            [pltpu.VMEM((CH,), jnp.int32) for _ in range(NF)],
            pltpu.VMEM((NF, CH, 128), jnp.float32),
            pltpu.VMEM((NF, CH, 128), jnp.float32),
            pltpu.SemaphoreType.DMA((2, NF)),
            pltpu.SemaphoreType.DMA((NF,)),
            pltpu.SemaphoreType.DMA((NF,)),
            pltpu.SemaphoreType.DMA((NF,)),
        ],
    )
    def k(Y_h, s_h, d_h, out_h, abufs, bbufs, A, B, semI, semA, semB, semS):
        c = lax.axis_index("c")
        s = lax.axis_index("s")
        wid = s * 2 + c

        @pl.loop(wid, nblk, step=2 * NSUB)
        def _(blk):
            base = blk * (NF * CH)
            ld = []
            for j in range(NF):
                ld.append(pltpu.async_copy(
                    s_h.at[pl.ds(base + j * CH, CH)], abufs[j],
                    semI.at[0, j]))
                ld.append(pltpu.async_copy(
                    d_h.at[pl.ds(base + j * CH, CH)], bbufs[j],
                    semI.at[1, j]))
            for d in ld:
                d.wait()
            for j in range(NF):
                for i in range(CH // 16):
                    bbufs[j][pl.ds(i * 16, 16)] = (
                        bbufs[j][pl.ds(i * 16, 16)] + n)
            gda = [pltpu.async_copy(Y_h.at[abufs[j]], A.at[j], semA.at[j])
                   for j in range(NF)]
            gdb = [pltpu.async_copy(Y_h.at[bbufs[j]], B.at[j], semB.at[j])
                   for j in range(NF)]
            sd = []
            for j in range(NF):
                gda[j].wait()
                gdb[j].wait()

                @pl.loop(0, CH)
                def _(r):
                    for i in range(8):
                        A[j, r, pl.ds(i * 16, 16)] = (
                            A[j, r, pl.ds(i * 16, 16)]
                            + B[j, r, pl.ds(i * 16, 16)])

                sd.append(pltpu.async_copy(
                    A.at[j], out_h.at[pl.ds(base + j * CH, CH)], semS.at[j]))
            for d in sd:
                d.wait()

    return k(Ycat, ssrc, sdst)


def kernel(input_feat, edge_index0, edge_index1, edge_type, edge_subg_index,
           rel_emb, W_out, b_out, W_in, b_in, W_g0, b_g0, W_g1, b_g1,
           W_fc, b_fc):
    n = input_feat.shape[0]
    E = edge_type.shape[0]
    ES = edge_subg_index.shape[1]

    T = _tc_rel_tables(rel_emb, W_out, b_out, W_in, b_in)
    nodes0 = edge_index0.reshape(-1)
    nodes1 = edge_index1.reshape(-1)
    T_big = jnp.tile(T, (CH, 1))
    nd_cat, d1_cat = _sc_embed(T_big, edge_type, nodes0, nodes1, n, E)

    xa, xb, nd0, ns1, nd1 = _tc_build_x(input_feat, nd_cat, d1_cat, n)
    xs_cat = jnp.concatenate([xa, xb], axis=0)

    agg0 = _sc_conv(xs_cat, edge_index0[0], edge_index0[1], n, E, 80, NB=2, ch=320)
    x1a, x1b = _tc_layer(agg0, nd0, ns1, W_g0, b_g0, n, 80, 256, True)
    x1s_cat = jnp.concatenate([x1a, x1b], axis=0)

    agg1 = _sc_conv(x1s_cat, edge_index1[0], edge_index1[1], n, E, 128, NB=2)
    x2a, x2b = _tc_layer(agg1, nd1, nd1, W_g1, b_g1, n, 128, 256, False)
    x2_cat = jnp.concatenate([x2a, x2b], axis=0)

    Wcat = jnp.concatenate([W_fc[:256], W_fc[256:]], axis=1)
    bcat = jnp.concatenate([b_fc, jnp.zeros_like(b_fc)], axis=0)
    Yt, Yb = _tc_fc(x2_cat, Wcat, bcat, n)
    Ycat = jnp.concatenate([Yt, Yb], axis=0)

    return _sc_final(Ycat, edge_subg_index[0], edge_subg_index[1], n, ES)


# speedup vs baseline: 1.7094x; 1.0346x over previous
import functools

import jax
import jax.numpy as jnp
from jax import lax
from jax.experimental import pallas as pl
from jax.experimental.pallas import tpu as pltpu
from jax.experimental.pallas import tpu_sc as plsc

CH = 128
NB = 4
RB = 80
NSUB = 16


def _mesh():
    return plsc.VectorSubcoreMesh(core_axis_name="c", subcore_axis_name="s")


_SC_PARAMS = pltpu.CompilerParams(use_tc_tiling_on_sc=False)


def _tc_rel_tables(rel_emb, W_out, b_out, W_in, b_in):
    def body(re_ref, wo_ref, bo_ref, wi_ref, bi_ref, out_ref):
        re = re_ref[:]
        ro = jnp.dot(re, wo_ref[:], preferred_element_type=jnp.float32) + bo_ref[:]
        ri = jnp.dot(re, wi_ref[:], preferred_element_type=jnp.float32) + bi_ref[:]
        ones = jnp.ones((16, 1), jnp.float32)
        zpad = jnp.zeros((16, 15), jnp.float32)
        out_ref[:] = jnp.concatenate(
            [jnp.concatenate([ro, ones, zpad], axis=1),
             jnp.concatenate([ri, ones, zpad], axis=1)], axis=0)

    return pl.pallas_call(
        body, out_shape=jax.ShapeDtypeStruct((32, 48), jnp.float32),
    )(rel_emb, W_out, b_out.reshape(1, -1), W_in, b_in.reshape(1, -1))


def _tc_build_x(input_feat, nd_cat, d1_cat, n):
    B = 1000
    grid = n // B

    def body(f_ref, ndo_ref, ndi_ref, d1o_ref, d1i_ref,
             xs_ref, nd0_ref, ns1_ref, nd1_ref):
        ndo = ndo_ref[:]
        ndi = ndi_ref[:]
        deg_o = ndo[:, 32:33]
        deg_i = ndi[:, 32:33]
        ns0 = jnp.where(deg_o > 0, lax.rsqrt(deg_o), 0.0)
        nd0_ref[:] = jnp.where(deg_i > 0, lax.rsqrt(deg_i), 0.0)
        d1o = d1o_ref[:, 0:1]
        d1i = d1i_ref[:, 0:1]
        ns1_ref[:] = jnp.where(d1o > 0, lax.rsqrt(d1o), 0.0)
        nd1_ref[:] = jnp.where(d1i > 0, lax.rsqrt(d1i), 0.0)
        rel = ndo[:, :32] + ndi[:, :32]
        xs_ref[:] = jnp.concatenate([f_ref[:], rel], axis=1) * ns0

    f32 = jnp.float32
    return pl.pallas_call(
        body,
        grid=(grid,),
        in_specs=[
            pl.BlockSpec((B, 128), lambda i: (i, 0)),
            pl.BlockSpec((B, 48), lambda i: (i, 0)),
            pl.BlockSpec((B, 48), lambda i, g=grid: (i + g, 0)),
            pl.BlockSpec((B, 16), lambda i: (i, 0)),
            pl.BlockSpec((B, 16), lambda i, g=grid: (i + g, 0)),
        ],
        out_specs=[
            pl.BlockSpec((B, 160), lambda i: (i, 0)),
            pl.BlockSpec((B, 1), lambda i: (i, 0)),
            pl.BlockSpec((B, 1), lambda i: (i, 0)),
            pl.BlockSpec((B, 1), lambda i: (i, 0)),
        ],
        out_shape=[
            jax.ShapeDtypeStruct((n, 160), f32),
            jax.ShapeDtypeStruct((n, 1), f32),
            jax.ShapeDtypeStruct((n, 1), f32),
            jax.ShapeDtypeStruct((n, 1), f32),
        ],
    )(input_feat, nd_cat, nd_cat, d1_cat, d1_cat)


def _tc_layer0(agg_cat, nd0, ns1, W, b, n):
    B = 1000
    grid = n // B

    def body(aa_ref, ab_ref, nd_ref, ns_ref, w_ref, b_ref, o_ref):
        h = (jnp.dot(aa_ref[:], w_ref[:80, :],
                     preferred_element_type=jnp.float32)
             + jnp.dot(ab_ref[:], w_ref[80:, :],
                       preferred_element_type=jnp.float32))
        x = jnp.maximum(h * nd_ref[:] + b_ref[:], 0.0)
        o_ref[:] = x * ns_ref[:]

    return pl.pallas_call(
        body,
        grid=(grid,),
        in_specs=[
            pl.BlockSpec((B, 80), lambda i: (i, 0)),
            pl.BlockSpec((B, 80), lambda i, g=grid: (i + g, 0)),
            pl.BlockSpec((B, 1), lambda i: (i, 0)),
            pl.BlockSpec((B, 1), lambda i: (i, 0)),
            pl.BlockSpec((160, 256), lambda i: (0, 0)),
            pl.BlockSpec((1, 256), lambda i: (0, 0)),
        ],
        out_specs=pl.BlockSpec((B, 256), lambda i: (i, 0)),
        out_shape=jax.ShapeDtypeStruct((n, 256), jnp.float32),
    )(agg_cat, agg_cat, nd0, ns1, W, b.reshape(1, -1))


def _tc_layer1_fc(agg_cat, nd1, W, b, Wcat, bcat, n):
    B = 1000
    grid = n // B

    def body(aa_ref, ab_ref, nd_ref, w_ref, b_ref, wc_ref, bc_ref, o_ref):
        h = (jnp.dot(aa_ref[:], w_ref[:128, :],
                     preferred_element_type=jnp.float32)
             + jnp.dot(ab_ref[:], w_ref[128:, :],
                       preferred_element_type=jnp.float32))
        x2 = jnp.maximum(h * nd_ref[:] + b_ref[:], 0.0)
        o_ref[:] = jnp.dot(x2, wc_ref[:],
                           preferred_element_type=jnp.float32) + bc_ref[:]

    return pl.pallas_call(
        body,
        grid=(grid,),
        in_specs=[
            pl.BlockSpec((B, 128), lambda i: (i, 0)),
            pl.BlockSpec((B, 128), lambda i, g=grid: (i + g, 0)),
            pl.BlockSpec((B, 1), lambda i: (i, 0)),
            pl.BlockSpec((256, 256), lambda i: (0, 0)),
            pl.BlockSpec((1, 256), lambda i: (0, 0)),
            pl.BlockSpec((256, 256), lambda i: (0, 0)),
            pl.BlockSpec((1, 256), lambda i: (0, 0)),
        ],
        out_specs=pl.BlockSpec((B, 256), lambda i: (i, 0)),
        out_shape=jax.ShapeDtypeStruct((n, 256), jnp.float32),
    )(agg_cat, agg_cat, nd1, W, b.reshape(1, -1), Wcat, bcat.reshape(1, -1))


def _sc_embed(T, etype, nodes0, nodes1, n, E):
    nblk = E // (CH * NB)
    nzb = n // RB
    zeros48 = jnp.zeros((RB, 48), jnp.float32)
    zeros16 = jnp.zeros((RB, 16), jnp.float32)
    ones16 = jnp.ones((CH, 16), jnp.float32)

    @functools.partial(
        pl.kernel,
        out_type=(jax.ShapeDtypeStruct((2 * n, 48), jnp.float32),
                  jax.ShapeDtypeStruct((2 * n, 16), jnp.float32)),
        mesh=_mesh(),
        compiler_params=_SC_PARAMS,
        scratch_types=[
            [pltpu.VMEM((CH,), jnp.int32) for _ in range(NB)],
            [pltpu.VMEM((CH,), jnp.int32) for _ in range(NB)],
            [pltpu.VMEM((CH,), jnp.int32) for _ in range(NB)],
            pltpu.VMEM((NB, CH, 48), jnp.float32),
            pltpu.VMEM((CH, 16), jnp.float32),
            pltpu.VMEM_SHARED((n, 48), jnp.float32),
            pltpu.VMEM_SHARED((n, 16), jnp.float32),
            pltpu.SemaphoreType.DMA((3, NB)),
            pltpu.SemaphoreType.DMA((NB,)),
            pltpu.SemaphoreType.DMA((NB,)),
            pltpu.SemaphoreType.DMA((NB,)),
        ],
    )
    def k(T_h, et_h, n0_h, n1_h, z48_h, z16_h, o16_h, outA_h, outB_h,
          tbufs, nbufs, n1bufs, rows, onesb, accA, accB,
          semI, semG, semS, semO):
        c = lax.axis_index("c")
        s = lax.axis_index("s")

        @pl.loop(s, nzb, step=NSUB)
        def _(b):
            pltpu.sync_copy(z48_h, accA.at[pl.ds(b * RB, RB)])
            pltpu.sync_copy(z16_h, accB.at[pl.ds(b * RB, RB)])

        pltpu.sync_copy(o16_h, onesb)
        plsc.subcore_barrier()

        eoff = c * E
        toff = c * 16

        @pl.loop(s, nblk, step=NSUB)
        def _(blk):
            base = blk * (NB * CH)
            ld = []
            for j in range(NB):
                ld.append(pltpu.async_copy(
                    et_h.at[pl.ds(base + j * CH, CH)], tbufs[j],
                    semI.at[0, j]))
                ld.append(pltpu.async_copy(
                    n0_h.at[pl.ds(eoff + base + j * CH, CH)], nbufs[j],
                    semI.at[1, j]))
                ld.append(pltpu.async_copy(
                    n1_h.at[pl.ds(eoff + base + j * CH, CH)], n1bufs[j],
                    semI.at[2, j]))
            for d in ld:
                d.wait()
            off16 = lax.iota(jnp.int32, 16) * 32
            for j in range(NB):
                for i in range(CH // 16):
                    tbufs[j][pl.ds(i * 16, 16)] = (
                        tbufs[j][pl.ds(i * 16, 16)] + (toff + 512 * i)
                        + off16)
            gd = [pltpu.async_copy(T_h.at[tbufs[j]], rows.at[j], semG.at[j])
                  for j in range(NB)]
            sd = []
            for j in range(NB):
                gd[j].wait()
                sd.append(pltpu.async_copy(rows.at[j], accA.at[nbufs[j]],
                                           semS.at[j], add=True))
                sd.append(pltpu.async_copy(onesb, accB.at[n1bufs[j]],
                                           semO.at[j], add=True))
            for d in sd:
                d.wait()

        plsc.subcore_barrier()
        noff = c * n

        @pl.loop(s, nzb, step=NSUB)
        def _(b):
            pltpu.sync_copy(accA.at[pl.ds(b * RB, RB)],
                            outA_h.at[pl.ds(noff + b * RB, RB)])
            pltpu.sync_copy(accB.at[pl.ds(b * RB, RB)],
                            outB_h.at[pl.ds(noff + b * RB, RB)])

    return k(T, etype, nodes0, nodes1, zeros48, zeros16, ones16)


def _sc_conv(xs_v, edges, n, E, W2, NB=NB, ch=CH):
    nblk = E // (ch * NB)
    nzb = n // RB
    zerosW = jnp.zeros((RB, W2), jnp.float32)

    @functools.partial(
        pl.kernel,
        out_type=jax.ShapeDtypeStruct((2 * n, W2), jnp.float32),
        mesh=_mesh(),
        compiler_params=_SC_PARAMS,
        scratch_types=[
            [pltpu.VMEM((ch,), jnp.int32) for _ in range(NB)],
            [pltpu.VMEM((ch,), jnp.int32) for _ in range(NB)],
            pltpu.VMEM((NB, ch, W2), jnp.float32),
            pltpu.VMEM_SHARED((n, W2), jnp.float32),
            pltpu.SemaphoreType.DMA((2, NB)),
            pltpu.SemaphoreType.DMA((NB,)),
            pltpu.SemaphoreType.DMA((NB,)),
        ],
    )
    def k(xs_h, e_h, zW_h, out_h, sbufs, dbufs, rows, acc, semI, semG, semS):
        c = lax.axis_index("c")
        s = lax.axis_index("s")

        @pl.loop(s, nzb, step=NSUB)
        def _(b):
            pltpu.sync_copy(zW_h, acc.at[pl.ds(b * RB, RB)])

        plsc.subcore_barrier()
        noff = c * n

        @pl.loop(s, nblk, step=NSUB)
        def _(blk):
            base = blk * (NB * ch)
            ld = []
            for j in range(NB):
                ld.append(pltpu.async_copy(
                    e_h.at[0, pl.ds(base + j * ch, ch)], sbufs[j],
                    semI.at[0, j]))
                ld.append(pltpu.async_copy(
                    e_h.at[1, pl.ds(base + j * ch, ch)], dbufs[j],
                    semI.at[1, j]))
            for d in ld:
                d.wait()
            for j in range(NB):
                for i in range(ch // 16):
                    sbufs[j][pl.ds(i * 16, 16)] = (
                        sbufs[j][pl.ds(i * 16, 16)] * 2 + c)
            gd = [pltpu.async_copy(xs_h.at[sbufs[j]], rows.at[j], semG.at[j])
                  for j in range(NB)]
            sd = []
            for j in range(NB):
                gd[j].wait()
                sd.append(pltpu.async_copy(rows.at[j], acc.at[dbufs[j]],
                                           semS.at[j], add=True))
            for d in sd:
                d.wait()

        plsc.subcore_barrier()

        @pl.loop(s, nzb, step=NSUB)
        def _(b):
            pltpu.sync_copy(acc.at[pl.ds(b * RB, RB)],
                            out_h.at[pl.ds(noff + b * RB, RB)])

    return k(xs_v, edges, zerosW)


def _sc_final(Y_v, subg, n, ES):
    NF = 2
    nblk = ES // (CH * NF)

    @functools.partial(
        pl.kernel,
        out_type=jax.ShapeDtypeStruct((ES, 128), jnp.float32),
        mesh=_mesh(),
        compiler_params=_SC_PARAMS,
        scratch_types=[
            [pltpu.VMEM((CH,), jnp.int32) for _ in range(NF)],
            [pltpu.VMEM((CH,), jnp.int32) for _ in range(NF)],
            pltpu.VMEM((NF, CH, 128), jnp.float32),
            pltpu.VMEM((NF, CH, 128), jnp.float32),
            pltpu.SemaphoreType.DMA((2, NF)),
            pltpu.SemaphoreType.DMA((NF,)),
            pltpu.SemaphoreType.DMA((NF,)),
            pltpu.SemaphoreType.DMA((NF,)),
        ],
    )
    def k(Y_h, e_h, out_h, abufs, bbufs, A, B, semI, semA, semB, semS):
        c = lax.axis_index("c")
        s = lax.axis_index("s")
        wid = s * 2 + c

        @pl.loop(wid, nblk, step=2 * NSUB)
        def _(blk):
            base = blk * (NF * CH)
            ld = []
            for j in range(NF):
                ld.append(pltpu.async_copy(
                    e_h.at[0, pl.ds(base + j * CH, CH)], abufs[j],
                    semI.at[0, j]))
                ld.append(pltpu.async_copy(
                    e_h.at[1, pl.ds(base + j * CH, CH)], bbufs[j],
                    semI.at[1, j]))
            for d in ld:
                d.wait()
            for j in range(NF):
                for i in range(CH // 16):
                    abufs[j][pl.ds(i * 16, 16)] = (
                        abufs[j][pl.ds(i * 16, 16)] * 2)
                    bbufs[j][pl.ds(i * 16, 16)] = (
                        bbufs[j][pl.ds(i * 16, 16)] * 2 + 1)
            gda = [pltpu.async_copy(Y_h.at[abufs[j]], A.at[j], semA.at[j])
                   for j in range(NF)]
            gdb = [pltpu.async_copy(Y_h.at[bbufs[j]], B.at[j], semB.at[j])
                   for j in range(NF)]
            sd = []
            for j in range(NF):
                gda[j].wait()
                gdb[j].wait()

                @pl.loop(0, CH)
                def _(r):
                    for i in range(8):
                        A[j, r, pl.ds(i * 16, 16)] = (
                            A[j, r, pl.ds(i * 16, 16)]
                            + B[j, r, pl.ds(i * 16, 16)])

                sd.append(pltpu.async_copy(
                    A.at[j], out_h.at[pl.ds(base + j * CH, CH)], semS.at[j]))
            for d in sd:
                d.wait()

    return k(Y_v, subg)


def kernel(input_feat, edge_index0, edge_index1, edge_type, edge_subg_index,
           rel_emb, W_out, b_out, W_in, b_in, W_g0, b_g0, W_g1, b_g1,
           W_fc, b_fc):
    n = input_feat.shape[0]
    E = edge_type.shape[0]
    ES = edge_subg_index.shape[1]

    T = _tc_rel_tables(rel_emb, W_out, b_out, W_in, b_in)
    T_big = jnp.tile(T, (CH, 1))
    nodes0 = edge_index0.reshape(-1)
    nodes1 = edge_index1.reshape(-1)
    nd_cat, d1_cat = _sc_embed(T_big, edge_type, nodes0, nodes1, n, E)

    xs, nd0, ns1, nd1 = _tc_build_x(input_feat, nd_cat, d1_cat, n)

    agg0 = _sc_conv(xs.reshape(2 * n, 80), edge_index0, n, E, 80)
    x1s = _tc_layer0(agg0, nd0, ns1, W_g0, b_g0, n)

    agg1 = _sc_conv(x1s.reshape(2 * n, 128), edge_index1, n, E, 128, NB=2)
    Wcat = jnp.concatenate([W_fc[:256], W_fc[256:]], axis=1)
    bcat = jnp.concatenate([b_fc, jnp.zeros_like(b_fc)], axis=0)
    Y = _tc_layer1_fc(agg1, nd1, W_g1, b_g1, Wcat, bcat, n)

    return _sc_final(Y.reshape(2 * n, 128), edge_subg_index, n, ES)


# speedup vs baseline: 1.7095x; 1.0001x over previous
import functools

import jax
import jax.numpy as jnp
from jax import lax
from jax.experimental import pallas as pl
from jax.experimental.pallas import tpu as pltpu
from jax.experimental.pallas import tpu_sc as plsc

CH = 128
NB = 4
RB = 80
NSUB = 16


def _mesh():
    return plsc.VectorSubcoreMesh(core_axis_name="c", subcore_axis_name="s")


_SC_PARAMS = pltpu.CompilerParams(use_tc_tiling_on_sc=False)


def _tc_rel_tables(rel_emb, W_out, b_out, W_in, b_in):
    def body(re_ref, wo_ref, bo_ref, wi_ref, bi_ref, out_ref):
        re = re_ref[:]
        ro = jnp.dot(re, wo_ref[:], preferred_element_type=jnp.float32) + bo_ref[:]
        ri = jnp.dot(re, wi_ref[:], preferred_element_type=jnp.float32) + bi_ref[:]
        ones = jnp.ones((16, 1), jnp.float32)
        zpad = jnp.zeros((16, 15), jnp.float32)
        out_ref[:] = jnp.concatenate(
            [jnp.concatenate([ro, ones, zpad], axis=1),
             jnp.concatenate([ri, ones, zpad], axis=1)], axis=0)

    return pl.pallas_call(
        body, out_shape=jax.ShapeDtypeStruct((32, 48), jnp.float32),
    )(rel_emb, W_out, b_out.reshape(1, -1), W_in, b_in.reshape(1, -1))


def _tc_build_x(input_feat, nd_cat, d1_cat, n):
    B = 1000
    grid = n // B

    def body(f_ref, ndo_ref, ndi_ref, d1o_ref, d1i_ref,
             xs_ref, nd0_ref, ns1_ref, nd1_ref):
        j = pl.program_id(0)
        ndo = ndo_ref[:]
        ndi = ndi_ref[:]
        deg_o = ndo[:, 32:33]
        deg_i = ndi[:, 32:33]
        ns0 = jnp.where(deg_o > 0, lax.rsqrt(deg_o), 0.0)
        nd0_ref[:] = jnp.where(deg_i > 0, lax.rsqrt(deg_i), 0.0)
        d1o = d1o_ref[:, 0:1]
        d1i = d1i_ref[:, 0:1]
        ns1_ref[:] = jnp.where(d1o > 0, lax.rsqrt(d1o), 0.0)
        nd1_ref[:] = jnp.where(d1i > 0, lax.rsqrt(d1i), 0.0)
        feat = f_ref[:]
        rel = ndo[:, :32] + ndi[:, :32]
        half = jnp.where(j == 0, feat[:, :80],
                         jnp.concatenate([feat[:, 80:], rel], axis=1))
        xs_ref[:] = half * ns0

    f32 = jnp.float32
    return pl.pallas_call(
        body,
        grid=(2, grid),
        in_specs=[
            pl.BlockSpec((B, 128), lambda j, i: (i, 0)),
            pl.BlockSpec((B, 48), lambda j, i: (i, 0)),
            pl.BlockSpec((B, 48), lambda j, i, g=grid: (i + g, 0)),
            pl.BlockSpec((B, 16), lambda j, i: (i, 0)),
            pl.BlockSpec((B, 16), lambda j, i, g=grid: (i + g, 0)),
        ],
        out_specs=[
            pl.BlockSpec((B, 80), lambda j, i, g=grid: (i + j * g, 0)),
            pl.BlockSpec((B, 1), lambda j, i: (i, 0)),
            pl.BlockSpec((B, 1), lambda j, i: (i, 0)),
            pl.BlockSpec((B, 1), lambda j, i: (i, 0)),
        ],
        out_shape=[
            jax.ShapeDtypeStruct((2 * n, 80), f32),
            jax.ShapeDtypeStruct((n, 1), f32),
            jax.ShapeDtypeStruct((n, 1), f32),
            jax.ShapeDtypeStruct((n, 1), f32),
        ],
    )(input_feat, nd_cat, nd_cat, d1_cat, d1_cat)


def _tc_layer0(agg_cat, nd0, ns1, W, b, n):
    B = 1000
    grid = n // B

    def body(aa_ref, ab_ref, nd_ref, ns_ref, w_ref, b_ref, o_ref):
        h = (jnp.dot(aa_ref[:], w_ref[:80, :],
                     preferred_element_type=jnp.float32)
             + jnp.dot(ab_ref[:], w_ref[80:, :],
                       preferred_element_type=jnp.float32))
        x = jnp.maximum(h * nd_ref[:] + b_ref[:], 0.0)
        o_ref[:] = x * ns_ref[:]

    return pl.pallas_call(
        body,
        grid=(2, grid),
        in_specs=[
            pl.BlockSpec((B, 80), lambda j, i: (i, 0)),
            pl.BlockSpec((B, 80), lambda j, i, g=grid: (i + g, 0)),
            pl.BlockSpec((B, 1), lambda j, i: (i, 0)),
            pl.BlockSpec((B, 1), lambda j, i: (i, 0)),
            pl.BlockSpec((160, 128), lambda j, i: (0, j)),
            pl.BlockSpec((1, 128), lambda j, i: (0, j)),
        ],
        out_specs=pl.BlockSpec((B, 128), lambda j, i, g=grid: (i + j * g, 0)),
        out_shape=jax.ShapeDtypeStruct((2 * n, 128), jnp.float32),
    )(agg_cat, agg_cat, nd0, ns1, W, b.reshape(1, -1))


def _tc_layer1_fc(agg_cat, nd1, W, b, Wcat, bcat, n):
    B = 1000
    grid = n // B

    def body(aa_ref, ab_ref, nd_ref, w_ref, b_ref, wc_ref, bc_ref,
             ot_ref, ob_ref):
        h = (jnp.dot(aa_ref[:], w_ref[:128, :],
                     preferred_element_type=jnp.float32)
             + jnp.dot(ab_ref[:], w_ref[128:, :],
                       preferred_element_type=jnp.float32))
        x2 = jnp.maximum(h * nd_ref[:] + b_ref[:], 0.0)
        y = jnp.dot(x2, wc_ref[:],
                    preferred_element_type=jnp.float32) + bc_ref[:]
        ot_ref[:] = y[:, :128]
        ob_ref[:] = y[:, 128:]

    f32 = jnp.float32
    return pl.pallas_call(
        body,
        grid=(grid,),
        in_specs=[
            pl.BlockSpec((B, 128), lambda i: (i, 0)),
            pl.BlockSpec((B, 128), lambda i, g=grid: (i + g, 0)),
            pl.BlockSpec((B, 1), lambda i: (i, 0)),
            pl.BlockSpec((256, 256), lambda i: (0, 0)),
            pl.BlockSpec((1, 256), lambda i: (0, 0)),
            pl.BlockSpec((256, 256), lambda i: (0, 0)),
            pl.BlockSpec((1, 256), lambda i: (0, 0)),
        ],
        out_specs=[
            pl.BlockSpec((B, 128), lambda i: (i, 0)),
            pl.BlockSpec((B, 128), lambda i: (i, 0)),
        ],
        out_shape=[
            jax.ShapeDtypeStruct((n, 128), f32),
            jax.ShapeDtypeStruct((n, 128), f32),
        ],
    )(agg_cat, agg_cat, nd1, W, b.reshape(1, -1), Wcat, bcat.reshape(1, -1))


def _sc_embed(T, etype, nodes0, nodes1, n, E):
    nblk = E // (CH * NB)
    nzb = n // RB
    zeros48 = jnp.zeros((RB, 48), jnp.float32)
    zeros16 = jnp.zeros((RB, 16), jnp.float32)
    ones16 = jnp.ones((CH, 16), jnp.float32)

    @functools.partial(
        pl.kernel,
        out_type=(jax.ShapeDtypeStruct((2 * n, 48), jnp.float32),
                  jax.ShapeDtypeStruct((2 * n, 16), jnp.float32)),
        mesh=_mesh(),
        compiler_params=_SC_PARAMS,
        scratch_types=[
            [pltpu.VMEM((CH,), jnp.int32) for _ in range(NB)],
            [pltpu.VMEM((CH,), jnp.int32) for _ in range(NB)],
            [pltpu.VMEM((CH,), jnp.int32) for _ in range(NB)],
            pltpu.VMEM((NB, CH, 48), jnp.float32),
            pltpu.VMEM((CH, 16), jnp.float32),
            pltpu.VMEM_SHARED((n, 48), jnp.float32),
            pltpu.VMEM_SHARED((n, 16), jnp.float32),
            pltpu.SemaphoreType.DMA((3, NB)),
            pltpu.SemaphoreType.DMA((NB,)),
            pltpu.SemaphoreType.DMA((NB,)),
            pltpu.SemaphoreType.DMA((NB,)),
        ],
    )
    def k(T_h, et_h, n0_h, n1_h, z48_h, z16_h, o16_h, outA_h, outB_h,
          tbufs, nbufs, n1bufs, rows, onesb, accA, accB,
          semI, semG, semS, semO):
        c = lax.axis_index("c")
        s = lax.axis_index("s")

        @pl.loop(s, nzb, step=NSUB)
        def _(b):
            pltpu.sync_copy(z48_h, accA.at[pl.ds(b * RB, RB)])
            pltpu.sync_copy(z16_h, accB.at[pl.ds(b * RB, RB)])

        pltpu.sync_copy(o16_h, onesb)
        plsc.subcore_barrier()

        eoff = c * E
        toff = c * 16

        @pl.loop(s, nblk, step=NSUB)
        def _(blk):
            base = blk * (NB * CH)
            ld = []
            for j in range(NB):
                ld.append(pltpu.async_copy(
                    et_h.at[pl.ds(base + j * CH, CH)], tbufs[j],
                    semI.at[0, j]))
                ld.append(pltpu.async_copy(
                    n0_h.at[pl.ds(eoff + base + j * CH, CH)], nbufs[j],
                    semI.at[1, j]))
                ld.append(pltpu.async_copy(
                    n1_h.at[pl.ds(eoff + base + j * CH, CH)], n1bufs[j],
                    semI.at[2, j]))
            for d in ld:
                d.wait()
            off16 = lax.iota(jnp.int32, 16) * 32
            for j in range(NB):
                for i in range(CH // 16):
                    tbufs[j][pl.ds(i * 16, 16)] = (
                        tbufs[j][pl.ds(i * 16, 16)] + (toff + 512 * i)
                        + off16)
            gd = [pltpu.async_copy(T_h.at[tbufs[j]], rows.at[j], semG.at[j])
                  for j in range(NB)]
            sd = []
            for j in range(NB):
                gd[j].wait()
                sd.append(pltpu.async_copy(rows.at[j], accA.at[nbufs[j]],
                                           semS.at[j], add=True))
                sd.append(pltpu.async_copy(onesb, accB.at[n1bufs[j]],
                                           semO.at[j], add=True))
            for d in sd:
                d.wait()

        plsc.subcore_barrier()
        noff = c * n

        @pl.loop(s, nzb, step=NSUB)
        def _(b):
            pltpu.sync_copy(accA.at[pl.ds(b * RB, RB)],
                            outA_h.at[pl.ds(noff + b * RB, RB)])
            pltpu.sync_copy(accB.at[pl.ds(b * RB, RB)],
                            outB_h.at[pl.ds(noff + b * RB, RB)])

    return k(T, etype, nodes0, nodes1, zeros48, zeros16, ones16)


def _sc_conv(xs_v, edges, n, E, W2, NB=NB, ch=CH):
    nblk = E // (ch * NB)
    nzb = n // RB
    zerosW = jnp.zeros((RB, W2), jnp.float32)

    @functools.partial(
        pl.kernel,
        out_type=jax.ShapeDtypeStruct((2 * n, W2), jnp.float32),
        mesh=_mesh(),
        compiler_params=_SC_PARAMS,
        scratch_types=[
            [pltpu.VMEM((ch,), jnp.int32) for _ in range(NB)],
            [pltpu.VMEM((ch,), jnp.int32) for _ in range(NB)],
            pltpu.VMEM((NB, ch, W2), jnp.float32),
            pltpu.VMEM_SHARED((n, W2), jnp.float32),
            pltpu.SemaphoreType.DMA((2, NB)),
            pltpu.SemaphoreType.DMA((NB,)),
            pltpu.SemaphoreType.DMA((NB,)),
        ],
    )
    def k(xs_h, e_h, zW_h, out_h, sbufs, dbufs, rows, acc, semI, semG, semS):
        c = lax.axis_index("c")
        s = lax.axis_index("s")
        goff = c * n

        @pl.loop(s, nzb, step=NSUB)
        def _(b):
            pltpu.sync_copy(zW_h, acc.at[pl.ds(b * RB, RB)])

        plsc.subcore_barrier()
        noff = c * n

        @pl.loop(s, nblk, step=NSUB)
        def _(blk):
            base = blk * (NB * ch)
            ld = []
            for j in range(NB):
                ld.append(pltpu.async_copy(
                    e_h.at[0, pl.ds(base + j * ch, ch)], sbufs[j],
                    semI.at[0, j]))
                ld.append(pltpu.async_copy(
                    e_h.at[1, pl.ds(base + j * ch, ch)], dbufs[j],
                    semI.at[1, j]))
            for d in ld:
                d.wait()
            for j in range(NB):
                for i in range(ch // 16):
                    sbufs[j][pl.ds(i * 16, 16)] = (
                        sbufs[j][pl.ds(i * 16, 16)] + goff)
            gd = [pltpu.async_copy(xs_h.at[sbufs[j]], rows.at[j], semG.at[j])
                  for j in range(NB)]
            sd = []
            for j in range(NB):
                gd[j].wait()
                sd.append(pltpu.async_copy(rows.at[j], acc.at[dbufs[j]],
                                           semS.at[j], add=True))
            for d in sd:
                d.wait()

        plsc.subcore_barrier()

        @pl.loop(s, nzb, step=NSUB)
        def _(b):
            pltpu.sync_copy(acc.at[pl.ds(b * RB, RB)],
                            out_h.at[pl.ds(noff + b * RB, RB)])

    return k(xs_v, edges, zerosW)


def _sc_final(Yt, Yb, subg, n, ES):
    NF = 2
    nblk = ES // (CH * NF)

    @functools.partial(
        pl.kernel,
        out_type=jax.ShapeDtypeStruct((ES, 128), jnp.float32),
        mesh=_mesh(),
        compiler_params=_SC_PARAMS,
        scratch_types=[
            [pltpu.VMEM((CH,), jnp.int32) for _ in range(NF)],
            [pltpu.VMEM((CH,), jnp.int32) for _ in range(NF)],
            pltpu.VMEM((NF, CH, 128), jnp.float32),
            pltpu.VMEM((NF, CH, 128), jnp.float32),
            pltpu.SemaphoreType.DMA((2, NF)),
            pltpu.SemaphoreType.DMA((NF,)),
            pltpu.SemaphoreType.DMA((NF,)),
            pltpu.SemaphoreType.DMA((NF,)),
        ],
    )
    def k(Yt_h, Yb_h, e_h, out_h, abufs, bbufs, A, B, semI, semA, semB, semS):
        c = lax.axis_index("c")
        s = lax.axis_index("s")
        wid = s * 2 + c

        @pl.loop(wid, nblk, step=2 * NSUB)
        def _(blk):
            base = blk * (NF * CH)
            ld = []
            for j in range(NF):
                ld.append(pltpu.async_copy(
                    e_h.at[0, pl.ds(base + j * CH, CH)], abufs[j],
                    semI.at[0, j]))
                ld.append(pltpu.async_copy(
                    e_h.at[1, pl.ds(base + j * CH, CH)], bbufs[j],
                    semI.at[1, j]))
            for d in ld:
                d.wait()
            gda = [pltpu.async_copy(Yt_h.at[abufs[j]], A.at[j], semA.at[j])
                   for j in range(NF)]
            gdb = [pltpu.async_copy(Yb_h.at[bbufs[j]], B.at[j], semB.at[j])
                   for j in range(NF)]
            sd = []
            for j in range(NF):
                gda[j].wait()
                gdb[j].wait()

                @pl.loop(0, CH)
                def _(r):
                    for i in range(8):
                        A[j, r, pl.ds(i * 16, 16)] = (
                            A[j, r, pl.ds(i * 16, 16)]
                            + B[j, r, pl.ds(i * 16, 16)])

                sd.append(pltpu.async_copy(
                    A.at[j], out_h.at[pl.ds(base + j * CH, CH)], semS.at[j]))
            for d in sd:
                d.wait()

    return k(Yt, Yb, subg)


def kernel(input_feat, edge_index0, edge_index1, edge_type, edge_subg_index,
           rel_emb, W_out, b_out, W_in, b_in, W_g0, b_g0, W_g1, b_g1,
           W_fc, b_fc):
    n = input_feat.shape[0]
    E = edge_type.shape[0]
    ES = edge_subg_index.shape[1]

    T = _tc_rel_tables(rel_emb, W_out, b_out, W_in, b_in)
    T_big = jnp.tile(T, (CH, 1))
    nodes0 = edge_index0.reshape(-1)
    nodes1 = edge_index1.reshape(-1)
    nd_cat, d1_cat = _sc_embed(T_big, edge_type, nodes0, nodes1, n, E)

    xs_cat, nd0, ns1, nd1 = _tc_build_x(input_feat, nd_cat, d1_cat, n)

    agg0 = _sc_conv(xs_cat, edge_index0, n, E, 80)
    x1s_cat = _tc_layer0(agg0, nd0, ns1, W_g0, b_g0, n)

    agg1 = _sc_conv(x1s_cat, edge_index1, n, E, 128, NB=2)
    Wcat = jnp.concatenate([W_fc[:256], W_fc[256:]], axis=1)
    bcat = jnp.concatenate([b_fc, jnp.zeros_like(b_fc)], axis=0)
    Yt, Yb = _tc_layer1_fc(agg1, nd1, W_g1, b_g1, Wcat, bcat, n)

    return _sc_final(Yt, Yb, edge_subg_index, n, ES)


# speedup vs baseline: 1.7658x; 1.0329x over previous
import functools

import jax
import jax.numpy as jnp
from jax import lax
from jax.experimental import pallas as pl
from jax.experimental.pallas import tpu as pltpu
from jax.experimental.pallas import tpu_sc as plsc

CH = 128
NB = 4
RB = 80
NSUB = 16


def _mesh():
    return plsc.VectorSubcoreMesh(core_axis_name="c", subcore_axis_name="s")


_SC_PARAMS = pltpu.CompilerParams(use_tc_tiling_on_sc=False)
_SC_PARAMS_NL = pltpu.CompilerParams(use_tc_tiling_on_sc=False,
                                     needs_layout_passes=False)


def _tc_rel_tables(rel_emb, W_out, b_out, W_in, b_in):
    def body(re_ref, wo_ref, bo_ref, wi_ref, bi_ref, out_ref):
        re = re_ref[:]
        ro = jnp.dot(re, wo_ref[:], preferred_element_type=jnp.float32) + bo_ref[:]
        ri = jnp.dot(re, wi_ref[:], preferred_element_type=jnp.float32) + bi_ref[:]
        out_ref[:] = jnp.concatenate([ro, ri], axis=0)

    return pl.pallas_call(
        body, out_shape=jax.ShapeDtypeStruct((32, 32), jnp.float32),
    )(rel_emb, W_out, b_out.reshape(1, -1), W_in, b_in.reshape(1, -1))


def _tc_build_x(input_feat, cnt_cat, d1_cat, RT, n):
    B = 1000
    grid = n // B

    def body(f_ref, co_ref, ci_ref, d1o_ref, d1i_ref, rt_ref,
             xs_ref, nd0_ref, ns1_ref, nd1_ref):
        j = pl.program_id(0)
        co = co_ref[:]
        ci = ci_ref[:]
        deg_o = jnp.sum(co, axis=1, keepdims=True)
        deg_i = jnp.sum(ci, axis=1, keepdims=True)
        ns0 = jnp.where(deg_o > 0, lax.rsqrt(deg_o), 0.0)
        nd0_ref[:] = jnp.where(deg_i > 0, lax.rsqrt(deg_i), 0.0)
        d1o = jnp.sum(d1o_ref[:], axis=1, keepdims=True) * (1.0 / 16.0)
        d1i = jnp.sum(d1i_ref[:], axis=1, keepdims=True) * (1.0 / 16.0)
        ns1_ref[:] = jnp.where(d1o > 0, lax.rsqrt(d1o), 0.0)
        nd1_ref[:] = jnp.where(d1i > 0, lax.rsqrt(d1i), 0.0)
        rel = (jnp.dot(co, rt_ref[:16, :], preferred_element_type=jnp.float32)
               + jnp.dot(ci, rt_ref[16:, :],
                         preferred_element_type=jnp.float32))
        feat = f_ref[:]
        half = jnp.where(j == 0, feat[:, :80],
                         jnp.concatenate([feat[:, 80:], rel], axis=1))
        xs_ref[:] = half * ns0

    f32 = jnp.float32
    return pl.pallas_call(
        body,
        grid=(2, grid),
        in_specs=[
            pl.BlockSpec((B, 128), lambda j, i: (i, 0)),
            pl.BlockSpec((B, 16), lambda j, i: (i, 0)),
            pl.BlockSpec((B, 16), lambda j, i, g=grid: (i + g, 0)),
            pl.BlockSpec((B, 16), lambda j, i: (i, 0)),
            pl.BlockSpec((B, 16), lambda j, i, g=grid: (i + g, 0)),
            pl.BlockSpec((32, 32), lambda j, i: (0, 0)),
        ],
        out_specs=[
            pl.BlockSpec((B, 80), lambda j, i, g=grid: (i + j * g, 0)),
            pl.BlockSpec((B, 1), lambda j, i: (i, 0)),
            pl.BlockSpec((B, 1), lambda j, i: (i, 0)),
            pl.BlockSpec((B, 1), lambda j, i: (i, 0)),
        ],
        out_shape=[
            jax.ShapeDtypeStruct((2 * n, 80), f32),
            jax.ShapeDtypeStruct((n, 1), f32),
            jax.ShapeDtypeStruct((n, 1), f32),
            jax.ShapeDtypeStruct((n, 1), f32),
        ],
    )(input_feat, cnt_cat, cnt_cat, d1_cat, d1_cat, RT)


def _tc_layer0(agg_cat, nd0, ns1, W, b, n):
    B = 1000
    grid = n // B

    def body(aa_ref, ab_ref, nd_ref, ns_ref, w_ref, b_ref, o_ref):
        h = (jnp.dot(aa_ref[:], w_ref[:80, :],
                     preferred_element_type=jnp.float32)
             + jnp.dot(ab_ref[:], w_ref[80:, :],
                       preferred_element_type=jnp.float32))
        x = jnp.maximum(h * nd_ref[:] + b_ref[:], 0.0)
        o_ref[:] = x * ns_ref[:]

    return pl.pallas_call(
        body,
        grid=(2, grid),
        in_specs=[
            pl.BlockSpec((B, 80), lambda j, i: (i, 0)),
            pl.BlockSpec((B, 80), lambda j, i, g=grid: (i + g, 0)),
            pl.BlockSpec((B, 1), lambda j, i: (i, 0)),
            pl.BlockSpec((B, 1), lambda j, i: (i, 0)),
            pl.BlockSpec((160, 128), lambda j, i: (0, j)),
            pl.BlockSpec((1, 128), lambda j, i: (0, j)),
        ],
        out_specs=pl.BlockSpec((B, 128), lambda j, i, g=grid: (i + j * g, 0)),
        out_shape=jax.ShapeDtypeStruct((2 * n, 128), jnp.float32),
    )(agg_cat, agg_cat, nd0, ns1, W, b.reshape(1, -1))


def _tc_layer1_fc(agg_cat, nd1, W, b, Wcat, bcat, n):
    B = 1000
    grid = n // B

    def body(aa_ref, ab_ref, nd_ref, w_ref, b_ref, wc_ref, bc_ref,
             ot_ref, ob_ref):
        h = (jnp.dot(aa_ref[:], w_ref[:128, :],
                     preferred_element_type=jnp.float32)
             + jnp.dot(ab_ref[:], w_ref[128:, :],
                       preferred_element_type=jnp.float32))
        x2 = jnp.maximum(h * nd_ref[:] + b_ref[:], 0.0)
        y = jnp.dot(x2, wc_ref[:],
                    preferred_element_type=jnp.float32) + bc_ref[:]
        ot_ref[:] = y[:, :128]
        ob_ref[:] = y[:, 128:]

    f32 = jnp.float32
    return pl.pallas_call(
        body,
        grid=(grid,),
        in_specs=[
            pl.BlockSpec((B, 128), lambda i: (i, 0)),
            pl.BlockSpec((B, 128), lambda i, g=grid: (i + g, 0)),
            pl.BlockSpec((B, 1), lambda i: (i, 0)),
            pl.BlockSpec((256, 256), lambda i: (0, 0)),
            pl.BlockSpec((1, 256), lambda i: (0, 0)),
            pl.BlockSpec((256, 256), lambda i: (0, 0)),
            pl.BlockSpec((1, 256), lambda i: (0, 0)),
        ],
        out_specs=[
            pl.BlockSpec((B, 128), lambda i: (i, 0)),
            pl.BlockSpec((B, 128), lambda i: (i, 0)),
        ],
        out_shape=[
            jax.ShapeDtypeStruct((n, 128), f32),
            jax.ShapeDtypeStruct((n, 128), f32),
        ],
    )(agg_cat, agg_cat, nd1, W, b.reshape(1, -1), Wcat, bcat.reshape(1, -1))


def _sc_embed(etype, nodes0, nodes1, n, E):
    nblk = E // (CH * NB)
    nzb = n // RB
    zeros16 = jnp.zeros((RB, 16), jnp.float32)
    ones16 = jnp.ones((CH, 16), jnp.float32)

    @functools.partial(
        pl.kernel,
        out_type=(jax.ShapeDtypeStruct((2 * n, 16), jnp.float32),
                  jax.ShapeDtypeStruct((2 * n, 16), jnp.float32)),
        mesh=_mesh(),
        compiler_params=_SC_PARAMS_NL,
        scratch_types=[
            [pltpu.VMEM((CH,), jnp.int32) for _ in range(NB)],
            [pltpu.VMEM((CH,), jnp.int32) for _ in range(NB)],
            [pltpu.VMEM((CH,), jnp.int32) for _ in range(NB)],
            [pltpu.VMEM((CH, 16), jnp.float32) for _ in range(NB)],
            pltpu.VMEM((CH, 16), jnp.float32),
            pltpu.VMEM_SHARED((n, 16), jnp.float32),
            pltpu.VMEM_SHARED((n, 16), jnp.float32),
            pltpu.SemaphoreType.DMA((3, NB)),
            pltpu.SemaphoreType.DMA((NB,)),
            pltpu.SemaphoreType.DMA((NB,)),
        ],
    )
    def k(et_h, n0_h, n1_h, z16_h, o16_h, outA_h, outB_h,
          tbufs, nbufs, n1bufs, ohbufs, onesb, accA, accB,
          semI, semS, semO):
        c = lax.axis_index("c")
        s = lax.axis_index("s")

        @pl.loop(s, nzb, step=NSUB)
        def _(b):
            pltpu.sync_copy(z16_h, accA.at[pl.ds(b * RB, RB)])
            pltpu.sync_copy(z16_h, accB.at[pl.ds(b * RB, RB)])

        pltpu.sync_copy(o16_h, onesb)
        plsc.subcore_barrier()

        eoff = c * E
        zv = jnp.zeros((16,), jnp.float32)
        onev = jnp.ones((16,), jnp.float32)
        rowid = lax.iota(jnp.int32, 16)

        @pl.loop(s, nblk, step=NSUB)
        def _(blk):
            base = blk * (NB * CH)
            ld = []
            for j in range(NB):
                ld.append(pltpu.async_copy(
                    et_h.at[pl.ds(base + j * CH, CH)], tbufs[j],
                    semI.at[0, j]))
                ld.append(pltpu.async_copy(
                    n0_h.at[pl.ds(eoff + base + j * CH, CH)], nbufs[j],
                    semI.at[1, j]))
                ld.append(pltpu.async_copy(
                    n1_h.at[pl.ds(eoff + base + j * CH, CH)], n1bufs[j],
                    semI.at[2, j]))
            sd = []
            for j in range(NB):
                ld[3 * j].wait()

                @pl.loop(0, CH)
                def _(r):
                    ohbufs[j][r, pl.ds(0, 16)] = zv

                for i in range(CH // 16):
                    tv = tbufs[j][pl.ds(i * 16, 16)]
                    plsc.store_scatter(ohbufs[j], [rowid + i * 16, tv], onev)
                ld[3 * j + 1].wait()
                ld[3 * j + 2].wait()
                sd.append(pltpu.async_copy(ohbufs[j], accA.at[nbufs[j]],
                                           semS.at[j], add=True))
                sd.append(pltpu.async_copy(onesb, accB.at[n1bufs[j]],
                                           semO.at[j], add=True))
            for d in sd:
                d.wait()

        plsc.subcore_barrier()
        noff = c * n

        @pl.loop(s, nzb, step=NSUB)
        def _(b):
            pltpu.sync_copy(accA.at[pl.ds(b * RB, RB)],
                            outA_h.at[pl.ds(noff + b * RB, RB)])
            pltpu.sync_copy(accB.at[pl.ds(b * RB, RB)],
                            outB_h.at[pl.ds(noff + b * RB, RB)])

    return k(etype, nodes0, nodes1, zeros16, ones16)


def _sc_conv(xs_v, edges, n, E, W2, NB=NB, ch=CH):
    nblk = E // (ch * NB)
    nzb = n // RB
    zerosW = jnp.zeros((RB, W2), jnp.float32)

    @functools.partial(
        pl.kernel,
        out_type=jax.ShapeDtypeStruct((2 * n, W2), jnp.float32),
        mesh=_mesh(),
        compiler_params=_SC_PARAMS,
        scratch_types=[
            [pltpu.VMEM((ch,), jnp.int32) for _ in range(NB)],
            [pltpu.VMEM((ch,), jnp.int32) for _ in range(NB)],
            pltpu.VMEM((NB, ch, W2), jnp.float32),
            pltpu.VMEM_SHARED((n, W2), jnp.float32),
            pltpu.SemaphoreType.DMA((2, NB)),
            pltpu.SemaphoreType.DMA((NB,)),
            pltpu.SemaphoreType.DMA((NB,)),
        ],
    )
    def k(xs_h, e_h, zW_h, out_h, sbufs, dbufs, rows, acc, semI, semG, semS):
        c = lax.axis_index("c")
        s = lax.axis_index("s")
        goff = c * n

        @pl.loop(s, nzb, step=NSUB)
        def _(b):
            pltpu.sync_copy(zW_h, acc.at[pl.ds(b * RB, RB)])

        plsc.subcore_barrier()
        noff = c * n

        @pl.loop(s, nblk, step=NSUB)
        def _(blk):
            base = blk * (NB * ch)
            ld = []
            for j in range(NB):
                ld.append(pltpu.async_copy(
                    e_h.at[0, pl.ds(base + j * ch, ch)], sbufs[j],
                    semI.at[0, j]))
                ld.append(pltpu.async_copy(
                    e_h.at[1, pl.ds(base + j * ch, ch)], dbufs[j],
                    semI.at[1, j]))
            for d in ld:
                d.wait()
            for j in range(NB):
                for i in range(ch // 16):
                    sbufs[j][pl.ds(i * 16, 16)] = (
                        sbufs[j][pl.ds(i * 16, 16)] + goff)
            gd = [pltpu.async_copy(xs_h.at[sbufs[j]], rows.at[j], semG.at[j])
                  for j in range(NB)]
            sd = []
            for j in range(NB):
                gd[j].wait()
                sd.append(pltpu.async_copy(rows.at[j], acc.at[dbufs[j]],
                                           semS.at[j], add=True))
            for d in sd:
                d.wait()

        plsc.subcore_barrier()

        @pl.loop(s, nzb, step=NSUB)
        def _(b):
            pltpu.sync_copy(acc.at[pl.ds(b * RB, RB)],
                            out_h.at[pl.ds(noff + b * RB, RB)])

    return k(xs_v, edges, zerosW)


def _sc_final(Yt, Yb, subg, n, ES):
    NF = 2
    nblk = ES // (CH * NF)

    @functools.partial(
        pl.kernel,
        out_type=jax.ShapeDtypeStruct((ES, 128), jnp.float32),
        mesh=_mesh(),
        compiler_params=_SC_PARAMS,
        scratch_types=[
            [pltpu.VMEM((CH,), jnp.int32) for _ in range(NF)],
            [pltpu.VMEM((CH,), jnp.int32) for _ in range(NF)],
            pltpu.VMEM((NF, CH, 128), jnp.float32),
            pltpu.VMEM((NF, CH, 128), jnp.float32),
            pltpu.SemaphoreType.DMA((2, NF)),
            pltpu.SemaphoreType.DMA((NF,)),
            pltpu.SemaphoreType.DMA((NF,)),
            pltpu.SemaphoreType.DMA((NF,)),
        ],
    )
    def k(Yt_h, Yb_h, e_h, out_h, abufs, bbufs, A, B, semI, semA, semB, semS):
        c = lax.axis_index("c")
        s = lax.axis_index("s")
        wid = s * 2 + c

        @pl.loop(wid, nblk, step=2 * NSUB)
        def _(blk):
            base = blk * (NF * CH)
            ld = []
            for j in range(NF):
                ld.append(pltpu.async_copy(
                    e_h.at[0, pl.ds(base + j * CH, CH)], abufs[j],
                    semI.at[0, j]))
                ld.append(pltpu.async_copy(
                    e_h.at[1, pl.ds(base + j * CH, CH)], bbufs[j],
                    semI.at[1, j]))
            for d in ld:
                d.wait()
            gda = [pltpu.async_copy(Yt_h.at[abufs[j]], A.at[j], semA.at[j])
                   for j in range(NF)]
            gdb = [pltpu.async_copy(Yb_h.at[bbufs[j]], B.at[j], semB.at[j])
                   for j in range(NF)]
            sd = []
            for j in range(NF):
                gda[j].wait()
                gdb[j].wait()

                @pl.loop(0, CH)
                def _(r):
                    for i in range(8):
                        A[j, r, pl.ds(i * 16, 16)] = (
                            A[j, r, pl.ds(i * 16, 16)]
                            + B[j, r, pl.ds(i * 16, 16)])

                sd.append(pltpu.async_copy(
                    A.at[j], out_h.at[pl.ds(base + j * CH, CH)], semS.at[j]))
            for d in sd:
                d.wait()

    return k(Yt, Yb, subg)


def kernel(input_feat, edge_index0, edge_index1, edge_type, edge_subg_index,
           rel_emb, W_out, b_out, W_in, b_in, W_g0, b_g0, W_g1, b_g1,
           W_fc, b_fc):
    n = input_feat.shape[0]
    E = edge_type.shape[0]
    ES = edge_subg_index.shape[1]

    RT = _tc_rel_tables(rel_emb, W_out, b_out, W_in, b_in)
    nodes0 = edge_index0.reshape(-1)
    nodes1 = edge_index1.reshape(-1)
    cnt_cat, d1_cat = _sc_embed(edge_type, nodes0, nodes1, n, E)

    xs_cat, nd0, ns1, nd1 = _tc_build_x(input_feat, cnt_cat, d1_cat, RT, n)

    agg0 = _sc_conv(xs_cat, edge_index0, n, E, 80)
    x1s_cat = _tc_layer0(agg0, nd0, ns1, W_g0, b_g0, n)

    agg1 = _sc_conv(x1s_cat, edge_index1, n, E, 128, NB=2)
    Wcat = jnp.concatenate([W_fc[:256], W_fc[256:]], axis=1)
    bcat = jnp.concatenate([b_fc, jnp.zeros_like(b_fc)], axis=0)
    Yt, Yb = _tc_layer1_fc(agg1, nd1, W_g1, b_g1, Wcat, bcat, n)

    return _sc_final(Yt, Yb, edge_subg_index, n, ES)
